# baseline (speedup 1.0000x reference)
import functools

import numpy as np
import jax
import jax.numpy as jnp
from jax import lax
from jax.experimental import pallas as pl
from jax.experimental.pallas import tpu as pltpu
from jax.experimental.pallas import tpu_sc as plsc

_B, _S, _D = 4096, 200, 64
_NC, _NS = 2, 16
_NW = _NC * _NS
_RPW = _B // _NW
_SCALE = 8.0
_HALF = _S // 2
_LANES = 16


def _make_pe() -> np.ndarray:
    pos = np.arange(_S, dtype=np.float32)[:, None]
    div = np.power(np.float32(10000.0),
                   np.arange(_D // 2, dtype=np.float32) * np.float32(2.0 / _D))
    pe = np.zeros((_S, _D), dtype=np.float32)
    pe[:, 0::2] = np.sin(pos / div)
    pe[:, 1::2] = np.cos(pos / div)
    return pe


_PE = _make_pe()

_mesh = plsc.VectorSubcoreMesh(core_axis_name="c", subcore_axis_name="s")


@functools.partial(
    pl.kernel,
    out_type=jax.ShapeDtypeStruct((_B * _S, _D), jnp.float32),
    mesh=_mesh,
    scratch_types=[
        pltpu.VMEM((2, _HALF), jnp.int32),
        pltpu.VMEM((_S, _D), jnp.float32),
        pltpu.VMEM((_S, _D), jnp.float32),
        pltpu.SemaphoreType.DMA,
    ],
    compiler_params=pltpu.CompilerParams(use_tc_tiling_on_sc=False),
)
def _embed(x_hbm, pe_hbm, table_hbm, out_hbm, idx_v, rows_v, pe_v, sem):
    wid = lax.axis_index("s") * _NC + lax.axis_index("c")
    pltpu.sync_copy(pe_hbm, pe_v)
    base_row = wid * _RPW

    @pl.loop(0, _RPW)
    def _row_loop(r):
        row = base_row + r
        pltpu.sync_copy(x_hbm.at[row], idx_v)
        g0 = pltpu.async_copy(table_hbm.at[idx_v.at[0]],
                              rows_v.at[pl.ds(0, _HALF)], sem)
        g1 = pltpu.async_copy(table_hbm.at[idx_v.at[1]],
                              rows_v.at[pl.ds(_HALF, _HALF)], sem)
        g0.wait()
        g1.wait()

        @pl.loop(0, _S)
        def _pos_loop(j):
            for k in range(_D // _LANES):
                sl = pl.ds(k * _LANES, _LANES)
                rows_v[j, sl] = rows_v[j, sl] * _SCALE + pe_v[j, sl]

        pltpu.sync_copy(rows_v, out_hbm.at[pl.ds(row * _S, _S)])


def kernel(x, table):
    xr = x.reshape(_B, 2, _HALF)
    pe = jnp.asarray(_PE)
    out = _embed(xr, pe, table)
    return out.reshape(_B, _S, _D)


# speedup vs baseline: 1.1963x; 1.1963x over previous
import functools

import numpy as np
import jax
import jax.numpy as jnp
from jax import lax
from jax.experimental import pallas as pl
from jax.experimental.pallas import tpu as pltpu
from jax.experimental.pallas import tpu_sc as plsc

_B, _S, _D = 4096, 200, 64
_NC, _NS = 2, 16
_NW = _NC * _NS
_RPW = _B // _NW
_SCALE = 8.0
_HALF = _S // 2
_LANES = 16
_CROWS = 2
_CT = _CROWS * _S
_NCH = _RPW // _CROWS
_NG = _CT // _HALF


def _make_pe() -> np.ndarray:
    pos = np.arange(_S, dtype=np.float32)[:, None]
    div = np.power(np.float32(10000.0),
                   np.arange(_D // 2, dtype=np.float32) * np.float32(2.0 / _D))
    pe = np.zeros((_S, _D), dtype=np.float32)
    pe[:, 0::2] = np.sin(pos / div)
    pe[:, 1::2] = np.cos(pos / div)
    return pe


_PE = _make_pe()

_mesh = plsc.VectorSubcoreMesh(core_axis_name="c", subcore_axis_name="s")


@functools.partial(
    pl.kernel,
    out_type=jax.ShapeDtypeStruct((_B * _S, _D), jnp.float32),
    mesh=_mesh,
    scratch_types=[
        pltpu.VMEM((_NG, _HALF), jnp.int32),
        pltpu.VMEM((_NG, _HALF), jnp.int32),
        pltpu.VMEM((_CT, _D), jnp.float32),
        pltpu.VMEM((_CT, _D), jnp.float32),
        pltpu.VMEM((_S, _D), jnp.float32),
        pltpu.SemaphoreType.DMA,
        pltpu.SemaphoreType.DMA,
        pltpu.SemaphoreType.DMA,
        pltpu.SemaphoreType.DMA,
        pltpu.SemaphoreType.DMA,
        pltpu.SemaphoreType.DMA,
    ],
    compiler_params=pltpu.CompilerParams(use_tc_tiling_on_sc=False),
)
def _embed(x_hbm, pe_hbm, table_hbm, out_hbm,
           idx0, idx1, rows0, rows1, pe_v,
           gsem0, gsem1, osem0, osem1, isem0, isem1):
    idx = (idx0, idx1)
    rows = (rows0, rows1)
    gsem = (gsem0, gsem1)
    osem = (osem0, osem1)
    isem = (isem0, isem1)
    wid = lax.axis_index("s") * _NC + lax.axis_index("c")
    pltpu.sync_copy(pe_hbm, pe_v)
    gbase = wid * _NCH

    def start_gathers(b):
        for h in range(_NG):
            pltpu.async_copy(table_hbm.at[idx[b].at[h]],
                             rows[b].at[pl.ds(h * _HALF, _HALF)], gsem[b])

    def wait_gathers(b):
        for h in range(_NG):
            pltpu.make_async_copy(table_hbm.at[idx[b].at[h]],
                                  rows[b].at[pl.ds(h * _HALF, _HALF)],
                                  gsem[b]).wait()

    pltpu.sync_copy(x_hbm.at[gbase], idx[0])
    start_gathers(0)
    pltpu.async_copy(x_hbm.at[gbase + 1], idx[1], isem[1])

    @pl.loop(0, _NCH, step=2)
    def _step(cc):
        for b in range(2):
            c = cc + b
            g = gbase + c
            nb = 1 - b

            @pl.when(c + 1 < _NCH)
            def _():
                @pl.when(c >= 1)
                def _():
                    pltpu.make_async_copy(
                        rows[nb], out_hbm.at[pl.ds((g - 1) * _CT, _CT)],
                        osem[nb]).wait()
                pltpu.make_async_copy(x_hbm.at[g + 1], idx[nb],
                                      isem[nb]).wait()
                start_gathers(nb)

            wait_gathers(b)

            @pl.when(c + 2 < _NCH)
            def _():
                pltpu.async_copy(x_hbm.at[g + 2], idx[b], isem[b])

            @plsc.parallel_loop(0, _S, unroll=2)
            def _compute(j):
                for half in range(_CROWS):
                    for k in range(_D // _LANES):
                        sl = pl.ds(k * _LANES, _LANES)
                        rows[b][half * _S + j, sl] = (
                            rows[b][half * _S + j, sl] * _SCALE + pe_v[j, sl])

            pltpu.async_copy(rows[b], out_hbm.at[pl.ds(g * _CT, _CT)], osem[b])

    for b, c in ((0, _NCH - 2), (1, _NCH - 1)):
        g = gbase + c
        pltpu.make_async_copy(rows[b], out_hbm.at[pl.ds(g * _CT, _CT)],
                              osem[b]).wait()


def kernel(x, table):
    xr = x.reshape(_B // _CROWS, _NG, _HALF)
    pe = jnp.asarray(_PE)
    out = _embed(xr, pe, table)
    return out.reshape(_B, _S, _D)


# speedup vs baseline: 1.2316x; 1.0295x over previous
import functools

import numpy as np
import jax
import jax.numpy as jnp
from jax import lax
from jax.experimental import pallas as pl
from jax.experimental.pallas import tpu as pltpu
from jax.experimental.pallas import tpu_sc as plsc

_B, _S, _D = 4096, 200, 64
_V = 1000000
_NC, _NS = 2, 16
_NW = _NC * _NS
_SCALE = 8.0
_LANES = 16

_BLKS = _V // 128
_TAIL = _V - _BLKS * 128
_APW = _BLKS // _NW
_AREM = _BLKS - _APW * _NW

_BC = _B // 128
_ITEMS = _S * _BC
_IPW = _ITEMS // _NW


def _make_pe() -> np.ndarray:
    pos = np.arange(_S, dtype=np.float32)[:, None]
    div = np.power(np.float32(10000.0),
                   np.arange(_D // 2, dtype=np.float32) * np.float32(2.0 / _D))
    pe = np.zeros((_S, _D), dtype=np.float32)
    pe[:, 0::2] = np.sin(pos / div)
    pe[:, 1::2] = np.cos(pos / div)
    return pe


_PE = _make_pe()

_mesh = plsc.VectorSubcoreMesh(core_axis_name="c", subcore_axis_name="s")


_AW = 2048
_AGRID = -(-_V // _AW)


def _detile_body(in_ref, out_ref):
    t3 = in_ref[...].T.reshape(_AW // 2, 2, _D)
    out_ref[...] = jnp.concatenate([t3[:, 0, :], t3[:, 1, :]], axis=1)


_detile = pl.pallas_call(
    _detile_body,
    out_shape=jax.ShapeDtypeStruct((_V // 2, 128), jnp.float32),
    grid=(_AGRID,),
    in_specs=[pl.BlockSpec((_D, _AW), lambda g: (0, g))],
    out_specs=pl.BlockSpec((_AW // 2, 128), lambda g: (g, 0)),
)


@functools.partial(
    pl.kernel,
    out_type=jax.ShapeDtypeStruct((_S, 8, _BC, 8, 128), jnp.float32),
    mesh=_mesh,
    scratch_types=[
        pltpu.VMEM((2, 128), jnp.int32),
        pltpu.VMEM((2, 128, _D), jnp.float32),
        pltpu.VMEM((2, 8, 8, 128), jnp.float32),
        pltpu.VMEM((_S, _D), jnp.float32),
        pltpu.SemaphoreType.DMA,
        pltpu.SemaphoreType.DMA,
        pltpu.SemaphoreType.DMA,
        pltpu.SemaphoreType.DMA,
        pltpu.SemaphoreType.DMA,
        pltpu.SemaphoreType.DMA,
    ],
    compiler_params=pltpu.CompilerParams(use_tc_tiling_on_sc=False,
                                         needs_layout_passes=False),
)
def _embed(xt_hbm, pe_hbm, tl_hbm, out_hbm, idx_v, gbuf, obuf, pe_v,
           isem0, isem1, gsem0, gsem1, osem0, osem1):
    isem = (isem0, isem1)
    gsem = (gsem0, gsem1)
    osem = (osem0, osem1)
    wid = lax.axis_index("s") * _NC + lax.axis_index("c")
    pltpu.sync_copy(pe_hbm, pe_v)
    mbase = wid * _IPW
    iota = jax.lax.iota(jnp.int32, _LANES)
    tok_idx = [iota + 16 * kk for kk in range(8)]

    def item_sb(i):
        m = mbase + i
        return m // _BC, m % _BC

    def start_idx(i, b):
        s, bc = item_sb(i)
        pltpu.async_copy(xt_hbm.at[s, pl.ds(bc * 128, 128)], idx_v.at[b],
                         isem[b])

    def wait_idx(b):
        pltpu.make_async_copy(xt_hbm.at[0, pl.ds(0, 128)], idx_v.at[b],
                              isem[b]).wait()

    def start_gather(b):
        pltpu.async_copy(tl_hbm.at[idx_v.at[b]], gbuf.at[b], gsem[b])

    def wait_gather(b):
        pltpu.make_async_copy(tl_hbm.at[idx_v.at[b]], gbuf.at[b],
                              gsem[b]).wait()

    def start_out(i, b):
        s, bc = item_sb(i)
        pltpu.async_copy(obuf.at[b], out_hbm.at[s, :, bc], osem[b])

    def wait_out(b):
        pltpu.make_async_copy(obuf.at[b], out_hbm.at[0, :, 0], osem[b]).wait()

    def compute(i, b):
        s, _ = item_sb(i)
        s_idx = jnp.full((_LANES,), s, jnp.int32)

        @plsc.parallel_loop(0, _D, unroll=2)
        def _dd(dd):
            dd_idx = jnp.full((_LANES,), dd, jnp.int32)
            pe_val = plsc.load_gather(pe_v, [s_idx, dd_idx])
            dr = dd // 8
            di = dd % 8
            for k in range(8):
                val = plsc.load_gather(gbuf.at[b], [tok_idx[k], dd_idx])
                obuf[b, dr, di, pl.ds(16 * k, 16)] = val * _SCALE + pe_val

    start_idx(0, 0)
    wait_idx(0)
    start_gather(0)
    start_idx(1, 1)

    @pl.loop(0, _IPW, step=2)
    def _item(ii):
        for b in range(2):
            i = ii + b
            nb = 1 - b

            @pl.when(i + 1 < _IPW)
            def _():
                wait_idx(nb)
                start_gather(nb)

            wait_gather(b)

            @pl.when(i + 2 < _IPW)
            def _():
                start_idx(i + 2, b)

            @pl.when(i >= 2)
            def _():
                wait_out(b)

            compute(i, b)
            start_out(i, b)

    wait_out(0)
    wait_out(1)


def kernel(x, table):
    tt = table.T
    t2 = _detile(tt)
    tl = t2.reshape(_V, _D)
    xt = x.T
    pe = jnp.asarray(_PE)
    out5 = _embed(xt, pe, tl)
    out = out5.transpose(2, 4, 0, 1, 3).reshape(_B, _S, _D)
    return out
